# baseline (speedup 1.0000x reference)
import functools

import jax
import jax.numpy as jnp
from jax import lax
from jax.experimental import pallas as pl
from jax.experimental.pallas import tpu as pltpu
from jax.experimental.pallas import tpu_sc as plsc

AVG_NUM_NEIGHBORS = 16.0


def _up_body(nf_ref, w_ref, out_ref):
    y = jnp.dot(nf_ref[...], w_ref[...], preferred_element_type=jnp.float32)
    c_half = y.shape[1] // 2
    out_ref[...] = jnp.stack([y[:, :c_half], y[:, c_half:]], axis=0)


def _linear_up(node_feats, w_up):
    n, c = node_feats.shape
    bn = 1000
    return pl.pallas_call(
        _up_body,
        grid=(n // bn,),
        in_specs=[
            pl.BlockSpec((bn, c), lambda i: (i, 0)),
            pl.BlockSpec((c, c), lambda i: (0, 0)),
        ],
        out_specs=pl.BlockSpec((2, bn, c // 2), lambda i: (0, i, 0)),
        out_shape=jax.ShapeDtypeStruct((2, n, c // 2), jnp.float32),
    )(node_feats, w_up)


_K = 32
_HALF = 64
_ACC_ROWS = 10000
_ROWS_PER_TEC = _ACC_ROWS // 16
_RB = _K // 8


def _msg_kernel_body(x2d_hbm, ef4_hbm, ea_hbm, sid2d_hbm, rid2d_hbm, zeros_hbm,
                     out_hbm, acc, sid_big, idx_big, ea_a, ea_b, ef0_a, ef0_b,
                     ef1_a, ef1_b, x_a, x_b, contrib_a, contrib_b, sem_a,
                     sem_b, sem_sa, sem_sb):
    c_idx = lax.axis_index("c")
    s_idx = lax.axis_index("s")
    n_nodes = x2d_hbm.shape[0] // 2
    total_blocks = sid2d_hbm.shape[0]
    n_extra = total_blocks % 16
    per_tec = total_blocks // 16
    first = s_idx * per_tec + jnp.minimum(s_idx, n_extra)
    cnt = per_tec + jnp.where(s_idx < n_extra, 1, 0)
    max_cnt = per_tec + (1 if n_extra else 0)
    n_pairs = per_tec // 2

    sets = ((ea_a, ef0_a, ef1_a, x_a, contrib_a, sem_a, sem_sa),
            (ea_b, ef0_b, ef1_b, x_b, contrib_b, sem_b, sem_sb))

    @pl.when(s_idx < n_extra)
    def _():
        pltpu.sync_copy(sid2d_hbm.at[pl.ds(first, max_cnt)],
                        sid_big.at[pl.ds(0, max_cnt)])
        pltpu.sync_copy(rid2d_hbm.at[pl.ds(first, max_cnt)],
                        idx_big.at[pl.ds(0, max_cnt)])

    @pl.when(s_idx >= n_extra)
    def _():
        pltpu.sync_copy(sid2d_hbm.at[pl.ds(first, per_tec)],
                        sid_big.at[pl.ds(0, per_tec)])
        pltpu.sync_copy(rid2d_hbm.at[pl.ds(first, per_tec)],
                        idx_big.at[pl.ds(0, per_tec)])

    @pl.loop(0, max_cnt)
    def _(r):
        @pl.loop(0, _K, step=16)
        def _(j):
            sid_big[r, pl.ds(j, 16)] = sid_big[r, pl.ds(j, 16)] + c_idx * n_nodes

    def issue(st, b, p):
        ea_s, ef0_s, ef1_s, x_s, _, sem, _ = sets[st]
        g = first + b
        pltpu.async_copy(ea_hbm.at[pl.ds(g * _K * 4, _K * 4)], ea_s, sem)
        if p == 0:
            pltpu.async_copy(
                ef4_hbm.at[pl.ds(g * _RB, _RB), 0, slice(None),
                           pl.ds(c_idx * _HALF, _HALF)],
                ef0_s, sem)
        pltpu.async_copy(
            ef4_hbm.at[pl.ds(g * _RB, _RB), 1, slice(None),
                       pl.ds(c_idx * _HALF, _HALF)],
            ef1_s, sem)
        pltpu.async_copy(x2d_hbm.at[sid_big.at[b]], x_s, sem)

    def drain(st, b, p):
        ea_s, ef0_s, ef1_s, x_s, _, sem, _ = sets[st]
        g = first + b
        pltpu.make_async_copy(ea_hbm.at[pl.ds(g * _K * 4, _K * 4)], ea_s,
                              sem).wait()
        if p == 0:
            pltpu.make_async_copy(
                ef4_hbm.at[pl.ds(g * _RB, _RB), 0, slice(None),
                           pl.ds(c_idx * _HALF, _HALF)],
                ef0_s, sem).wait()
        pltpu.make_async_copy(
            ef4_hbm.at[pl.ds(g * _RB, _RB), 1, slice(None),
                       pl.ds(c_idx * _HALF, _HALF)],
            ef1_s, sem).wait()
        pltpu.make_async_copy(x2d_hbm.at[sid_big.at[b]], x_s, sem).wait()

    def compute_scatter(st, b, p):
        ea_s, ef0_s, ef1_s, x_s, contrib, _, sem_s = sets[st]

        @pl.when(b >= 2)
        def _():
            pltpu.make_async_copy(contrib, acc.at[idx_big.at[b - 2]],
                                  sem_s).wait()

        @pl.loop(0, _RB)
        def _(rb):
            ea16a = ea_s[pl.ds(rb * 32, 16)]
            ea16b = ea_s[pl.ds(rb * 32 + 16, 16)]
            for r in range(8):
                kk = rb * 8 + r
                eav = ea16a if r < 4 else ea16b
                e0 = eav[(r % 4) * 4 + 2 * p]
                e1 = eav[(r % 4) * 4 + 2 * p + 1]
                for q in range(_HALF // 16):
                    sl = pl.ds(q * 16, 16)
                    xv = x_s[kk, sl]
                    u1 = xv * ef1_s[rb, r, sl]
                    if p == 0:
                        u0 = xv * ef0_s[rb, r, sl]
                        contrib[kk, sl] = u0 * e0
                    else:
                        contrib[kk, sl] = u1 * e0
                    contrib[kk, pl.ds(_HALF + q * 16, 16)] = u1 * e1

        pltpu.async_copy(contrib, acc.at[idx_big.at[b]], sem_s, add=True)

    for p in range(2):
        pltpu.sync_copy(zeros_hbm, acc.at[pl.ds(s_idx * _ROWS_PER_TEC,
                                                _ROWS_PER_TEC)])
        plsc.subcore_barrier()

        issue(0, 0, p)

        @pl.loop(0, n_pairs)
        def _(i):
            b0 = 2 * i
            issue(1, b0 + 1, p)
            drain(0, b0, p)
            compute_scatter(0, b0, p)

            @pl.when(b0 + 2 < cnt)
            def _():
                issue(0, b0 + 2, p)

            drain(1, b0 + 1, p)
            compute_scatter(1, b0 + 1, p)

        @pl.when(cnt > per_tec)
        def _():
            drain(0, per_tec, p)
            compute_scatter(0, per_tec, p)

        last_a = jnp.where(cnt > per_tec, per_tec, per_tec - 2)
        pltpu.make_async_copy(contrib_a, acc.at[idx_big.at[last_a]],
                              sem_sa).wait()
        pltpu.make_async_copy(contrib_b, acc.at[idx_big.at[per_tec - 1]],
                              sem_sb).wait()

        plsc.subcore_barrier()

        row0 = s_idx * _ROWS_PER_TEC
        for ml in range(2):
            m = 2 * p + ml
            pltpu.sync_copy(
                acc.at[pl.ds(row0, _ROWS_PER_TEC), pl.ds(ml * _HALF, _HALF)],
                out_hbm.at[pl.ds(row0, _ROWS_PER_TEC),
                           pl.ds(m * 128 + c_idx * _HALF, _HALF)])

        plsc.subcore_barrier()


def _messages(x2d, ef4, edge_attrs, sender, receiver, n_nodes):
    zeros = jnp.zeros((_ROWS_PER_TEC, 2 * _HALF), jnp.float32)
    n_edges = sender.shape[0]
    total_blocks = n_edges // _K
    max_cnt = total_blocks // 16 + (1 if total_blocks % 16 else 0)
    sid2d = sender.reshape(total_blocks, _K)
    rid2d = receiver.reshape(total_blocks, _K)
    mesh = plsc.VectorSubcoreMesh(core_axis_name="c", subcore_axis_name="s")
    kern = pl.kernel(
        _msg_kernel_body,
        out_type=jax.ShapeDtypeStruct((n_nodes, 512), jnp.float32),
        mesh=mesh,
        compiler_params=pltpu.CompilerParams(use_tc_tiling_on_sc=False),
        scratch_types=[
            pltpu.VMEM_SHARED((_ACC_ROWS, 2 * _HALF), jnp.float32),
            pltpu.VMEM((max_cnt, _K), jnp.int32),
            pltpu.VMEM((max_cnt, _K), jnp.int32),
            pltpu.VMEM((_K * 4,), jnp.float32),
            pltpu.VMEM((_K * 4,), jnp.float32),
            pltpu.VMEM((_RB, 8, _HALF), jnp.float32),
            pltpu.VMEM((_RB, 8, _HALF), jnp.float32),
            pltpu.VMEM((_RB, 8, _HALF), jnp.float32),
            pltpu.VMEM((_RB, 8, _HALF), jnp.float32),
            pltpu.VMEM((_K, _HALF), jnp.float32),
            pltpu.VMEM((_K, _HALF), jnp.float32),
            pltpu.VMEM((_K, 2 * _HALF), jnp.float32),
            pltpu.VMEM((_K, 2 * _HALF), jnp.float32),
            pltpu.SemaphoreType.DMA,
            pltpu.SemaphoreType.DMA,
            pltpu.SemaphoreType.DMA,
            pltpu.SemaphoreType.DMA,
        ],
    )
    return kern(x2d, ef4, edge_attrs.reshape(-1), sid2d, rid2d, zeros)


def _out_body(msg_ref, na_ref, wlin_ref, wskip_ref, out_ref):
    na = na_ref[...]
    z_dim = na.shape[1]
    inv = 1.0 / AVG_NUM_NEIGHBORS
    for m in range(4):
        l = 0 if m == 0 else 1
        ml = jnp.dot(msg_ref[:, m * 128:(m + 1) * 128].astype(jnp.bfloat16),
                     wlin_ref[l].astype(jnp.bfloat16),
                     preferred_element_type=jnp.float32) * inv
        acc = jnp.zeros((ml.shape[0], out_ref.shape[2]), jnp.float32)
        for z in range(z_dim):
            acc = acc + jnp.dot((ml * na[:, z:z + 1]).astype(jnp.bfloat16),
                                wskip_ref[l, :, z, :].astype(jnp.bfloat16),
                                preferred_element_type=jnp.float32)
        out_ref[:, m, :] = acc


def _skip_out(message, node_attrs, w_lin, w_skip):
    n = message.shape[0]
    c = message.shape[1] // 4
    z = node_attrs.shape[1]
    bn = 400
    return pl.pallas_call(
        _out_body,
        grid=(n // bn,),
        in_specs=[
            pl.BlockSpec((bn, 4 * c), lambda i: (i, 0)),
            pl.BlockSpec((bn, z), lambda i: (i, 0)),
            pl.BlockSpec((2, c, c), lambda i: (0, 0, 0)),
            pl.BlockSpec((2, c, z, c), lambda i: (0, 0, 0, 0)),
        ],
        out_specs=pl.BlockSpec((bn, 4, c), lambda i: (i, 0, 0)),
        out_shape=jax.ShapeDtypeStruct((n, 4, c), jnp.float32),
    )(message, node_attrs, w_lin, w_skip)


def kernel(node_attrs, node_feats, edge_attrs, edge_feats, edge_index, W_up,
           W_lin, W_skip):
    n, c = node_feats.shape
    e = edge_attrs.shape[0]
    sender = edge_index[0].astype(jnp.int32)
    receiver = edge_index[1].astype(jnp.int32)
    ef4 = edge_feats.reshape(e // 8, 8, 2, c).transpose(0, 2, 1, 3)

    x2 = _linear_up(node_feats, W_up)
    x2d = x2.reshape(2 * n, c // 2)
    message = _messages(x2d, ef4, edge_attrs, sender, receiver, n)
    return _skip_out(message, node_attrs, W_lin, W_skip)


# speedup vs baseline: 1.1709x; 1.1709x over previous
import functools

import jax
import jax.numpy as jnp
from jax import lax
from jax.experimental import pallas as pl
from jax.experimental.pallas import tpu as pltpu
from jax.experimental.pallas import tpu_sc as plsc

AVG_NUM_NEIGHBORS = 16.0


def _up_body(nf_ref, w_ref, out_ref):
    y = jnp.dot(nf_ref[...], w_ref[...], preferred_element_type=jnp.float32)
    c_half = y.shape[1] // 2
    out_ref[...] = jnp.stack([y[:, :c_half], y[:, c_half:]], axis=0)


def _linear_up(node_feats, w_up):
    n, c = node_feats.shape
    bn = 1000
    return pl.pallas_call(
        _up_body,
        grid=(n // bn,),
        in_specs=[
            pl.BlockSpec((bn, c), lambda i: (i, 0)),
            pl.BlockSpec((c, c), lambda i: (0, 0)),
        ],
        out_specs=pl.BlockSpec((2, bn, c // 2), lambda i: (0, i, 0)),
        out_shape=jax.ShapeDtypeStruct((2, n, c // 2), jnp.float32),
    )(node_feats, w_up)


_K = 32
_HALF = 64
_ACC_ROWS = 10000
_ROWS_PER_TEC = _ACC_ROWS // 16
_RB = _K // 8


def _msg_kernel_body(x2d_hbm, ef4_hbm, ea_t_hbm, sid2d_hbm, rid2d_hbm, zeros_hbm,
                     out_hbm, acc, sid_big, idx_big, ea_a, ea_b, ef0_a, ef0_b,
                     ef1_a, ef1_b, x_a, x_b, contrib_a, contrib_b, sem_a,
                     sem_b, sem_sa, sem_sb):
    c_idx = lax.axis_index("c")
    s_idx = lax.axis_index("s")
    n_nodes = x2d_hbm.shape[0] // 2
    total_blocks = sid2d_hbm.shape[0]
    n_extra = total_blocks % 16
    per_tec = total_blocks // 16
    first = s_idx * per_tec + jnp.minimum(s_idx, n_extra)
    cnt = per_tec + jnp.where(s_idx < n_extra, 1, 0)
    max_cnt = per_tec + (1 if n_extra else 0)
    n_pairs = per_tec // 2

    sets = ((ea_a, ef0_a, ef1_a, x_a, contrib_a, sem_a, sem_sa),
            (ea_b, ef0_b, ef1_b, x_b, contrib_b, sem_b, sem_sb))

    @pl.when(s_idx < n_extra)
    def _():
        pltpu.sync_copy(sid2d_hbm.at[pl.ds(first, max_cnt)],
                        sid_big.at[pl.ds(0, max_cnt)])
        pltpu.sync_copy(rid2d_hbm.at[pl.ds(first, max_cnt)],
                        idx_big.at[pl.ds(0, max_cnt)])

    @pl.when(s_idx >= n_extra)
    def _():
        pltpu.sync_copy(sid2d_hbm.at[pl.ds(first, per_tec)],
                        sid_big.at[pl.ds(0, per_tec)])
        pltpu.sync_copy(rid2d_hbm.at[pl.ds(first, per_tec)],
                        idx_big.at[pl.ds(0, per_tec)])

    @pl.loop(0, max_cnt)
    def _(r):
        @pl.loop(0, _K, step=16)
        def _(j):
            sid_big[r, pl.ds(j, 16)] = sid_big[r, pl.ds(j, 16)] + c_idx * n_nodes

    def issue(st, b, p):
        ea_s, ef0_s, ef1_s, x_s, _, sem, _ = sets[st]
        g = first + b
        pltpu.async_copy(x2d_hbm.at[sid_big.at[b]], x_s, sem)
        pltpu.async_copy(
            ea_t_hbm.at[g // 4, pl.ds(2 * p, 2), pl.ds((g % 4) * _K, _K)],
            ea_s.at[:, pl.ds(0, _K)], sem)
        if p == 0:
            pltpu.async_copy(
                ef4_hbm.at[pl.ds(g * _RB, _RB), 0, slice(None),
                           pl.ds(c_idx * _HALF, _HALF)],
                ef0_s, sem)
        pltpu.async_copy(
            ef4_hbm.at[pl.ds(g * _RB, _RB), 1, slice(None),
                       pl.ds(c_idx * _HALF, _HALF)],
            ef1_s, sem)

    def drain(st, b, p):
        ea_s, ef0_s, ef1_s, x_s, _, sem, _ = sets[st]
        g = first + b
        pltpu.make_async_copy(x2d_hbm.at[sid_big.at[b]], x_s, sem).wait()
        pltpu.make_async_copy(
            ea_t_hbm.at[g // 4, pl.ds(2 * p, 2), pl.ds((g % 4) * _K, _K)],
            ea_s.at[:, pl.ds(0, _K)], sem).wait()
        if p == 0:
            pltpu.make_async_copy(
                ef4_hbm.at[pl.ds(g * _RB, _RB), 0, slice(None),
                           pl.ds(c_idx * _HALF, _HALF)],
                ef0_s, sem).wait()
        pltpu.make_async_copy(
            ef4_hbm.at[pl.ds(g * _RB, _RB), 1, slice(None),
                       pl.ds(c_idx * _HALF, _HALF)],
            ef1_s, sem).wait()

    def compute_scatter(st, b, p):
        ea_s, ef0_s, ef1_s, x_s, contrib, _, sem_s = sets[st]

        @pl.when(b >= 2)
        def _():
            pltpu.make_async_copy(contrib, acc.at[idx_big.at[b - 2]],
                                  sem_s).wait()

        @pl.loop(0, _RB)
        def _(rb):
            ea0v = ea_s[0, pl.ds(rb * 8, 16)]
            ea1v = ea_s[1, pl.ds(rb * 8, 16)]
            for r in range(8):
                kk = rb * 8 + r
                e0 = ea0v[r]
                e1 = ea1v[r]
                for q in range(_HALF // 16):
                    sl = pl.ds(q * 16, 16)
                    xv = x_s[kk, sl]
                    u1 = xv * ef1_s[rb, r, sl]
                    if p == 0:
                        u0 = xv * ef0_s[rb, r, sl]
                        contrib[kk, sl] = u0 * e0
                    else:
                        contrib[kk, sl] = u1 * e0
                    contrib[kk, pl.ds(_HALF + q * 16, 16)] = u1 * e1

        pltpu.async_copy(contrib, acc.at[idx_big.at[b]], sem_s, add=True)

    for p in range(2):
        pltpu.sync_copy(zeros_hbm, acc.at[pl.ds(s_idx * _ROWS_PER_TEC,
                                                _ROWS_PER_TEC)])
        plsc.subcore_barrier()

        issue(0, 0, p)

        @pl.loop(0, n_pairs)
        def _(i):
            b0 = 2 * i
            issue(1, b0 + 1, p)
            drain(0, b0, p)
            compute_scatter(0, b0, p)

            @pl.when(b0 + 2 < cnt)
            def _():
                issue(0, b0 + 2, p)

            drain(1, b0 + 1, p)
            compute_scatter(1, b0 + 1, p)

        @pl.when(cnt > per_tec)
        def _():
            drain(0, per_tec, p)
            compute_scatter(0, per_tec, p)

        last_a = jnp.where(cnt > per_tec, per_tec, per_tec - 2)
        pltpu.make_async_copy(contrib_a, acc.at[idx_big.at[last_a]],
                              sem_sa).wait()
        pltpu.make_async_copy(contrib_b, acc.at[idx_big.at[per_tec - 1]],
                              sem_sb).wait()

        plsc.subcore_barrier()

        row0 = s_idx * _ROWS_PER_TEC
        for ml in range(2):
            m = 2 * p + ml
            pltpu.sync_copy(
                acc.at[pl.ds(row0, _ROWS_PER_TEC), pl.ds(ml * _HALF, _HALF)],
                out_hbm.at[pl.ds(row0, _ROWS_PER_TEC),
                           pl.ds(m * 128 + c_idx * _HALF, _HALF)])

        plsc.subcore_barrier()


def _messages(x2d, ef4, ea_t, sender, receiver, n_nodes):
    zeros = jnp.zeros((_ROWS_PER_TEC, 2 * _HALF), jnp.float32)
    n_edges = sender.shape[0]
    total_blocks = n_edges // _K
    max_cnt = total_blocks // 16 + (1 if total_blocks % 16 else 0)
    sid2d = sender.reshape(total_blocks, _K)
    rid2d = receiver.reshape(total_blocks, _K)
    mesh = plsc.VectorSubcoreMesh(core_axis_name="c", subcore_axis_name="s")
    kern = pl.kernel(
        _msg_kernel_body,
        out_type=jax.ShapeDtypeStruct((n_nodes, 512), jnp.float32),
        mesh=mesh,
        compiler_params=pltpu.CompilerParams(use_tc_tiling_on_sc=False),
        scratch_types=[
            pltpu.VMEM_SHARED((_ACC_ROWS, 2 * _HALF), jnp.float32),
            pltpu.VMEM((max_cnt, _K), jnp.int32),
            pltpu.VMEM((max_cnt, _K), jnp.int32),
            pltpu.VMEM((2, _K + 16), jnp.float32),
            pltpu.VMEM((2, _K + 16), jnp.float32),
            pltpu.VMEM((_RB, 8, _HALF), jnp.float32),
            pltpu.VMEM((_RB, 8, _HALF), jnp.float32),
            pltpu.VMEM((_RB, 8, _HALF), jnp.float32),
            pltpu.VMEM((_RB, 8, _HALF), jnp.float32),
            pltpu.VMEM((_K, _HALF), jnp.float32),
            pltpu.VMEM((_K, _HALF), jnp.float32),
            pltpu.VMEM((_K, 2 * _HALF), jnp.float32),
            pltpu.VMEM((_K, 2 * _HALF), jnp.float32),
            pltpu.SemaphoreType.DMA,
            pltpu.SemaphoreType.DMA,
            pltpu.SemaphoreType.DMA,
            pltpu.SemaphoreType.DMA,
        ],
    )
    return kern(x2d, ef4, ea_t, sid2d, rid2d, zeros)


def _out_body(msg_ref, na_ref, wlin_ref, wskip_ref, out_ref):
    na = na_ref[...]
    z_dim = na.shape[1]
    inv = 1.0 / AVG_NUM_NEIGHBORS
    for m in range(4):
        l = 0 if m == 0 else 1
        ml = jnp.dot(msg_ref[:, m * 128:(m + 1) * 128], wlin_ref[l],
                     preferred_element_type=jnp.float32) * inv
        acc = jnp.zeros((ml.shape[0], out_ref.shape[2]), jnp.float32)
        for z in range(z_dim):
            acc = acc + jnp.dot(ml * na[:, z:z + 1], wskip_ref[l, :, z, :],
                                preferred_element_type=jnp.float32)
        out_ref[:, m, :] = acc


def _skip_out(message, node_attrs, w_lin, w_skip):
    n = message.shape[0]
    c = message.shape[1] // 4
    z = node_attrs.shape[1]
    bn = 400
    return pl.pallas_call(
        _out_body,
        grid=(n // bn,),
        in_specs=[
            pl.BlockSpec((bn, 4 * c), lambda i: (i, 0)),
            pl.BlockSpec((bn, z), lambda i: (i, 0)),
            pl.BlockSpec((2, c, c), lambda i: (0, 0, 0)),
            pl.BlockSpec((2, c, z, c), lambda i: (0, 0, 0, 0)),
        ],
        out_specs=pl.BlockSpec((bn, 4, c), lambda i: (i, 0, 0)),
        out_shape=jax.ShapeDtypeStruct((n, 4, c), jnp.float32),
    )(message, node_attrs, w_lin, w_skip)


def kernel(node_attrs, node_feats, edge_attrs, edge_feats, edge_index, W_up,
           W_lin, W_skip):
    n, c = node_feats.shape
    e = edge_attrs.shape[0]
    sender = edge_index[0].astype(jnp.int32)
    receiver = edge_index[1].astype(jnp.int32)
    ef4 = edge_feats.reshape(e // 8, 8, 2, c).transpose(0, 2, 1, 3)
    ea_t = edge_attrs.T.reshape(4, e // 128, 128).transpose(1, 0, 2)

    x2 = _linear_up(node_feats, W_up)
    x2d = x2.reshape(2 * n, c // 2)
    message = _messages(x2d, ef4, ea_t, sender, receiver, n)
    return _skip_out(message, node_attrs, W_lin, W_skip)
